# Initial kernel scaffold; baseline (speedup 1.0000x reference)
import jax
import jax.numpy as jnp
from jax.experimental import pallas as pl


def kernel(x, edge_index, pestat, W_pe, W_self, W_nbr, b_env, Ain_self, Ain_nbr, Ain_out, Ain_bias, Aout_self, Aout_nbr, Aout_out, Aout_bias):
    raise NotImplementedError("write your pallas kernel here")


# speedup vs baseline: 7.9310x; 7.9310x over previous
import functools

import jax
import jax.numpy as jnp
from jax import lax
from jax.experimental import pallas as pl
from jax.experimental.pallas import tpu as pltpu
from jax.experimental.pallas import tpu_sc as plsc

N = 10000
E = 320000
D = 128
H = 16
TEMP = 0.5

NC = 2
NS = 16
NW = NC * NS
EPT = E // NW
CHUNK = 80
NCHUNK = EPT // CHUNK
NPAD = 10112
RPT = NPAD // NS


def _seg_sum_partials(table, stab, src, dst):
  mesh = plsc.VectorSubcoreMesh(core_axis_name="c", subcore_axis_name="s")

  @functools.partial(
      pl.kernel,
      out_type=(jax.ShapeDtypeStruct((NC, NPAD, D), jnp.float32),
                jax.ShapeDtypeStruct((NC * NPAD,), jnp.float32)),
      mesh=mesh,
      scratch_types=[
          pltpu.VMEM((CHUNK,), jnp.int32),
          pltpu.VMEM((CHUNK,), jnp.int32),
          pltpu.VMEM((CHUNK, D), jnp.float32),
          pltpu.VMEM((CHUNK,), jnp.float32),
          pltpu.VMEM((RPT,), jnp.float32),
          pltpu.VMEM_SHARED((NPAD, D), jnp.float32),
          pltpu.VMEM_SHARED((NPAD,), jnp.float32),
          pltpu.SemaphoreType.DMA,
      ],
  )
  def seg_kernel(table_hbm, stab_hbm, src_hbm, dst_hbm, outr_hbm, outs_hbm,
                 sidx, didx, rows, svals, zscal, acc, sacc, sem):
    c = lax.axis_index("c")
    s = lax.axis_index("s")

    z16 = jnp.zeros((16,), jnp.float32)

    def zrow(r, _):
      for j in range(D // 16):
        rows[r, pl.ds(j * 16, 16)] = z16
      return 0

    lax.fori_loop(0, CHUNK, zrow, 0)

    def zsc(r, _):
      zscal[pl.ds(r * 16, 16)] = z16
      return 0

    lax.fori_loop(0, RPT // 16, zsc, 0)
    zscal[pl.ds(RPT - 16, 16)] = z16

    row0 = s * RPT
    for k in range(RPT // CHUNK):
      pltpu.sync_copy(rows, acc.at[pl.ds(row0 + k * CHUNK, CHUNK)])
    rem = RPT % CHUNK
    if rem:
      pltpu.sync_copy(rows.at[pl.ds(0, rem)],
                      acc.at[pl.ds(row0 + (RPT // CHUNK) * CHUNK, rem)])
    pltpu.sync_copy(zscal, sacc.at[pl.ds(row0, RPT)])
    plsc.subcore_barrier()

    base = (s * NC + c) * EPT

    def body(j, _):
      off = base + j * CHUNK
      pltpu.sync_copy(src_hbm.at[pl.ds(off, CHUNK)], sidx)
      pltpu.sync_copy(dst_hbm.at[pl.ds(off, CHUNK)], didx)
      pltpu.async_copy(table_hbm.at[sidx], rows, sem).wait()
      pltpu.sync_copy(rows, acc.at[didx], add=True)
      pltpu.async_copy(stab_hbm.at[sidx], svals, sem).wait()
      pltpu.sync_copy(svals, sacc.at[didx], add=True)
      return 0

    lax.fori_loop(0, NCHUNK, body, 0)
    plsc.subcore_barrier()
    for k in range(RPT // CHUNK):
      pltpu.sync_copy(acc.at[pl.ds(row0 + k * CHUNK, CHUNK)], rows)
      pltpu.sync_copy(rows, outr_hbm.at[c, pl.ds(row0 + k * CHUNK, CHUNK)])
    if rem:
      r0 = row0 + (RPT // CHUNK) * CHUNK
      pltpu.sync_copy(acc.at[pl.ds(r0, rem)], rows.at[pl.ds(0, rem)])
      pltpu.sync_copy(rows.at[pl.ds(0, rem)], outr_hbm.at[c, pl.ds(r0, rem)])
    pltpu.sync_copy(sacc.at[pl.ds(row0, RPT)], zscal)
    pltpu.sync_copy(zscal, outs_hbm.at[pl.ds(c * NPAD + row0, RPT)])

  outr, outs = seg_kernel(table, stab, src, dst)
  return outr, outs.reshape(NC, NPAD)[:, :N]


def _embed_body(x_ref, pe_ref, wpe_ref, h_ref):
  h_ref[...] = x_ref[...] + jnp.dot(pe_ref[...], wpe_ref[...],
                                    preferred_element_type=jnp.float32)


def _gates_body(h_ref, sr_ref, ss_ref, ains_ref, ainn_ref, aino_ref, ainb_ref,
                aouts_ref, aoutn_ref, aouto_ref, aoutb_ref,
                ik_ref, ok_ref, gp_ref):
  h = h_ref[...]
  S = sr_ref[0, :N] + sr_ref[1, :N]
  deg = jnp.maximum(ss_ref[0] + ss_ref[1], 1.0)[:, None]
  m = S / deg

  def keep(a_s, a_n, a_o, a_b):
    z = jnp.maximum(jnp.dot(h, a_s, preferred_element_type=jnp.float32)
                    + jnp.dot(m, a_n, preferred_element_type=jnp.float32), 0.0)
    lo = jnp.dot(z, a_o, preferred_element_type=jnp.float32) + a_b
    return jax.nn.sigmoid((lo[:, 0:1] - lo[:, 1:2]) / TEMP)

  in_keep = keep(ains_ref[...], ainn_ref[...], aino_ref[...], ainb_ref[...])
  out_keep = keep(aouts_ref[...], aoutn_ref[...], aouto_ref[...], aoutb_ref[...])
  ik_ref[...] = in_keep
  ok_ref[...] = out_keep[:, 0]
  gp_ref[...] = out_keep * h


def _update_body(h_ref, sr_ref, ss_ref, ik_ref, ws_ref, wn_ref, b_ref, ho_ref):
  h = h_ref[...]
  S = sr_ref[0, :N] + sr_ref[1, :N]
  sw = (ss_ref[0] + ss_ref[1])[:, None]
  ik = ik_ref[...]
  m = (ik * S) / (ik * sw + 1e-8)
  ho_ref[...] = jnp.maximum(jnp.dot(h, ws_ref[...], preferred_element_type=jnp.float32)
                            + jnp.dot(m, wn_ref[...], preferred_element_type=jnp.float32)
                            + b_ref[...], 0.0) + h


def _tc(body, out_shape):
  return pl.pallas_call(body, out_shape=out_shape)


def kernel(x, edge_index, pestat, W_pe, W_self, W_nbr, b_env,
           Ain_self, Ain_nbr, Ain_out, Ain_bias,
           Aout_self, Aout_nbr, Aout_out, Aout_bias):
  src = edge_index[0].astype(jnp.int32)
  dst = edge_index[1].astype(jnp.int32)
  L = W_self.shape[0]

  h = _tc(_embed_body, jax.ShapeDtypeStruct((N, D), jnp.float32))(
      x, pestat, W_pe)

  ones_tab = jnp.ones((N,), jnp.float32)
  ainb = Ain_bias.reshape(1, 2)
  aoutb = Aout_bias.reshape(1, 2)

  for l in range(L):
    s0r, s0s = _seg_sum_partials(h, ones_tab, src, dst)
    in_keep, ok_tab, gp = _tc(
        _gates_body,
        [jax.ShapeDtypeStruct((N, 1), jnp.float32),
         jax.ShapeDtypeStruct((N,), jnp.float32),
         jax.ShapeDtypeStruct((N, D), jnp.float32)],
    )(h, s0r, s0s, Ain_self, Ain_nbr, Ain_out, ainb,
      Aout_self, Aout_nbr, Aout_out, aoutb)
    s1r, s1s = _seg_sum_partials(gp, ok_tab, src, dst)
    h = _tc(_update_body, jax.ShapeDtypeStruct((N, D), jnp.float32))(
        h, s1r, s1s, in_keep, W_self[l], W_nbr[l], b_env[l].reshape(1, D))

  return h


# speedup vs baseline: 18.9770x; 2.3928x over previous
import functools

import jax
import jax.numpy as jnp
from jax import lax
from jax.experimental import pallas as pl
from jax.experimental.pallas import tpu as pltpu
from jax.experimental.pallas import tpu_sc as plsc

N = 10000
E = 320000
D = 128
H = 16
TEMP = 0.5

NC = 2
NS = 16
NW = NC * NS
EPT = E // NW
CHUNK = 40
NCHUNK = EPT // CHUNK
NPAD = 10112
RPT = NPAD // NS

NB = 5
NI = 2 * NB
NG = NCHUNK // NB


def _seg_sum_partials(table, stab, src, dst, do_scal=True):
  mesh = plsc.VectorSubcoreMesh(core_axis_name="c", subcore_axis_name="s")

  out_type = [jax.ShapeDtypeStruct((NC, NPAD, D), jnp.float32)]
  if do_scal:
    out_type.append(jax.ShapeDtypeStruct((NC * NPAD,), jnp.float32))

  scratch = (
      [pltpu.VMEM((CHUNK,), jnp.int32) for _ in range(NI)]
      + [pltpu.VMEM((CHUNK,), jnp.int32) for _ in range(NI)]
      + [pltpu.VMEM((CHUNK, D), jnp.float32) for _ in range(NB)]
      + [pltpu.VMEM((CHUNK,), jnp.float32) for _ in range(NB)]
      + [pltpu.VMEM((RPT,), jnp.float32)]
      + [pltpu.VMEM_SHARED((NPAD, D), jnp.float32)]
      + [pltpu.VMEM_SHARED((NPAD,), jnp.float32)]
      + [pltpu.SemaphoreType.DMA for _ in range(NI + 2 * NB)]
  )

  @functools.partial(pl.kernel, out_type=tuple(out_type), mesh=mesh,
                     scratch_types=tuple(scratch))
  def seg_kernel(table_hbm, stab_hbm, src_hbm, dst_hbm, *rest):
    no = len(out_type)
    outr_hbm = rest[0]
    outs_hbm = rest[1] if do_scal else None
    sc = list(rest[no:])
    sidx = sc[:NI]
    didx = sc[NI:2 * NI]
    rows = sc[2 * NI:2 * NI + NB]
    svals = sc[2 * NI + NB:2 * NI + 2 * NB]
    zscal = sc[2 * NI + 2 * NB]
    acc = sc[2 * NI + 2 * NB + 1]
    sacc = sc[2 * NI + 2 * NB + 2]
    sems_all = sc[2 * NI + 2 * NB + 3:]
    semi = sems_all[:NI]
    semg = sems_all[NI:NI + NB]
    sems = sems_all[NI + NB:]

    c = lax.axis_index("c")
    s = lax.axis_index("s")
    z16 = jnp.zeros((16,), jnp.float32)

    def zrow(r, _):
      for j in range(D // 16):
        rows[0][r, pl.ds(j * 16, 16)] = z16
      return 0

    lax.fori_loop(0, CHUNK, zrow, 0)

    def zsc(r, _):
      zscal[pl.ds(r * 16, 16)] = z16
      return 0

    lax.fori_loop(0, RPT // 16, zsc, 0)
    zscal[pl.ds(RPT - 16, 16)] = z16

    row0 = s * RPT
    rem = RPT % CHUNK
    for k in range(RPT // CHUNK):
      pltpu.sync_copy(rows[0], acc.at[pl.ds(row0 + k * CHUNK, CHUNK)])
    if rem:
      pltpu.sync_copy(rows[0].at[pl.ds(0, rem)],
                      acc.at[pl.ds(row0 + (RPT // CHUNK) * CHUNK, rem)])
    if do_scal:
      pltpu.sync_copy(zscal, sacc.at[pl.ds(row0, RPT)])
    plsc.subcore_barrier()

    base = (s * NC + c) * EPT

    def idx_load(j, islot):
      off = base + j * CHUNK
      pltpu.make_async_copy(src_hbm.at[pl.ds(off, CHUNK)], sidx[islot],
                            semi[islot]).start()
      pltpu.make_async_copy(dst_hbm.at[pl.ds(off, CHUNK)], didx[islot],
                            semi[islot]).start()

    def idx_wait(islot):
      pltpu.make_async_copy(src_hbm.at[pl.ds(0, CHUNK)], sidx[islot],
                            semi[islot]).wait()
      pltpu.make_async_copy(dst_hbm.at[pl.ds(0, CHUNK)], didx[islot],
                            semi[islot]).wait()

    def gath(b, islot):
      return (pltpu.make_async_copy(table_hbm.at[sidx[islot]], rows[b], semg[b]),
              pltpu.make_async_copy(stab_hbm.at[sidx[islot]], svals[b], semg[b])
              if do_scal else None)

    def scat(b, islot):
      return (pltpu.make_async_copy(rows[b], acc.at[didx[islot]], sems[b]),
              pltpu.make_async_copy(svals[b], sacc.at[didx[islot]], sems[b])
              if do_scal else None)


    for b in range(NB):
      idx_load(b, b)

    def two_groups(gg, _):
      for half in range(2):
        g = gg * 2 + half
        lo = half * NB
        hi = (1 - half) * NB

        for b in range(NB):
          jb = g * NB + b

          @pl.when(g > 0)
          def _():
            ra, sa = scat(b, hi + b)
            ra.wait()
            if do_scal:
              sa.wait()

          @pl.when(g < NG - 1)
          def _():
            idx_load(jb + NB, hi + b)

        for b in range(NB):
          idx_wait(lo + b)
          ra, sa = gath(b, lo + b)
          ra.start()
          if do_scal:
            sa.start()

        for b in range(NB):
          ra, sa = gath(b, lo + b)
          ra.wait()
          if do_scal:
            sa.wait()
          rb, sb = scat(b, lo + b)
          rb.start(add=True)
          if do_scal:
            sb.start(add=True)
      return 0

    lax.fori_loop(0, NG // 2, two_groups, 0)
    last_lo = ((NG - 1) % 2) * NB
    for b in range(NB):
      ra, sa = scat(b, last_lo + b)
      ra.wait()
      if do_scal:
        sa.wait()

    plsc.subcore_barrier()
    for k in range(RPT // CHUNK):
      pltpu.sync_copy(acc.at[pl.ds(row0 + k * CHUNK, CHUNK)], rows[0])
      pltpu.sync_copy(rows[0], outr_hbm.at[c, pl.ds(row0 + k * CHUNK, CHUNK)])
    if rem:
      r0 = row0 + (RPT // CHUNK) * CHUNK
      pltpu.sync_copy(acc.at[pl.ds(r0, rem)], rows[0].at[pl.ds(0, rem)])
      pltpu.sync_copy(rows[0].at[pl.ds(0, rem)], outr_hbm.at[c, pl.ds(r0, rem)])
    if do_scal:
      pltpu.sync_copy(sacc.at[pl.ds(row0, RPT)], zscal)
      pltpu.sync_copy(zscal, outs_hbm.at[pl.ds(c * NPAD + row0, RPT)])

  if do_scal:
    outr, outs = seg_kernel(table, stab, src, dst)
    return outr, outs.reshape(NC, NPAD)[:, :N]
  (outr,) = seg_kernel(table, stab, src, dst)
  return outr, None


def _embed_body(x_ref, pe_ref, wpe_ref, h_ref):
  h_ref[...] = x_ref[...] + jnp.dot(pe_ref[...], wpe_ref[...],
                                    preferred_element_type=jnp.float32)


def _gates_body(h_ref, sr_ref, ss_ref, ains_ref, ainn_ref, aino_ref, ainb_ref,
                aouts_ref, aoutn_ref, aouto_ref, aoutb_ref,
                ik_ref, ok_ref, gp_ref):
  h = h_ref[...]
  S = sr_ref[0, :N] + sr_ref[1, :N]
  deg = jnp.maximum(ss_ref[0] + ss_ref[1], 1.0)[:, None]
  m = S / deg

  def keep(a_s, a_n, a_o, a_b):
    z = jnp.maximum(jnp.dot(h, a_s, preferred_element_type=jnp.float32)
                    + jnp.dot(m, a_n, preferred_element_type=jnp.float32), 0.0)
    lo = jnp.dot(z, a_o, preferred_element_type=jnp.float32) + a_b
    return jax.nn.sigmoid((lo[:, 0:1] - lo[:, 1:2]) / TEMP)

  in_keep = keep(ains_ref[...], ainn_ref[...], aino_ref[...], ainb_ref[...])
  out_keep = keep(aouts_ref[...], aoutn_ref[...], aouto_ref[...], aoutb_ref[...])
  ik_ref[...] = in_keep
  ok_ref[...] = out_keep[:, 0]
  gp_ref[...] = out_keep * h


def _update_body(h_ref, sr_ref, ss_ref, ik_ref, ws_ref, wn_ref, b_ref, ho_ref):
  h = h_ref[...]
  S = sr_ref[0, :N] + sr_ref[1, :N]
  sw = (ss_ref[0] + ss_ref[1])[:, None]
  ik = ik_ref[...]
  m = (ik * S) / (ik * sw + 1e-8)
  ho_ref[...] = jnp.maximum(jnp.dot(h, ws_ref[...], preferred_element_type=jnp.float32)
                            + jnp.dot(m, wn_ref[...], preferred_element_type=jnp.float32)
                            + b_ref[...], 0.0) + h


def _tc(body, out_shape):
  return pl.pallas_call(body, out_shape=out_shape)


def kernel(x, edge_index, pestat, W_pe, W_self, W_nbr, b_env,
           Ain_self, Ain_nbr, Ain_out, Ain_bias,
           Aout_self, Aout_nbr, Aout_out, Aout_bias):
  src = edge_index[0].astype(jnp.int32)
  dst = edge_index[1].astype(jnp.int32)
  L = W_self.shape[0]

  h = _tc(_embed_body, jax.ShapeDtypeStruct((N, D), jnp.float32))(
      x, pestat, W_pe)

  ones_tab = jnp.ones((N,), jnp.float32)
  ainb = Ain_bias.reshape(1, 2)
  aoutb = Aout_bias.reshape(1, 2)

  deg2 = None
  for l in range(L):
    s0r, s0s = _seg_sum_partials(h, ones_tab, src, dst, do_scal=(l == 0))
    if l == 0:
      deg2 = s0s
    in_keep, ok_tab, gp = _tc(
        _gates_body,
        [jax.ShapeDtypeStruct((N, 1), jnp.float32),
         jax.ShapeDtypeStruct((N,), jnp.float32),
         jax.ShapeDtypeStruct((N, D), jnp.float32)],
    )(h, s0r, deg2, Ain_self, Ain_nbr, Ain_out, ainb,
      Aout_self, Aout_nbr, Aout_out, aoutb)
    s1r, s1s = _seg_sum_partials(gp, ok_tab, src, dst)
    h = _tc(_update_body, jax.ShapeDtypeStruct((N, D), jnp.float32))(
        h, s1r, s1s, in_keep, W_self[l], W_nbr[l], b_env[l].reshape(1, D))

  return h


# speedup vs baseline: 19.3797x; 1.0212x over previous
import functools

import jax
import jax.numpy as jnp
from jax import lax
from jax.experimental import pallas as pl
from jax.experimental.pallas import tpu as pltpu
from jax.experimental.pallas import tpu_sc as plsc

N = 10000
E = 320000
D = 128
H = 16
TEMP = 0.5

NC = 2
NS = 16
NW = NC * NS
EPT = E // NW
CHUNK = 80
NCHUNK = EPT // CHUNK
NPAD = 10112
RPT = NPAD // NS

NB = 4
NI = 2 * NB
NGF = 30


def _seg_sum_partials(table, stab, src, dst, do_scal=True):
  mesh = plsc.VectorSubcoreMesh(core_axis_name="c", subcore_axis_name="s")

  out_type = [jax.ShapeDtypeStruct((NC, NPAD, D), jnp.float32)]
  if do_scal:
    out_type.append(jax.ShapeDtypeStruct((NC * NPAD,), jnp.float32))

  scratch = (
      [pltpu.VMEM((CHUNK,), jnp.int32) for _ in range(NI)]
      + [pltpu.VMEM((CHUNK,), jnp.int32) for _ in range(NI)]
      + [pltpu.VMEM((CHUNK, D), jnp.float32) for _ in range(NB)]
      + [pltpu.VMEM((CHUNK,), jnp.float32) for _ in range(NB)]
      + [pltpu.VMEM((RPT,), jnp.float32)]
      + [pltpu.VMEM_SHARED((NPAD, D), jnp.float32)]
      + [pltpu.VMEM_SHARED((NPAD,), jnp.float32)]
      + [pltpu.SemaphoreType.DMA for _ in range(NI + 2 * NB)]
  )

  @functools.partial(pl.kernel, out_type=tuple(out_type), mesh=mesh,
                     scratch_types=tuple(scratch))
  def seg_kernel(table_hbm, stab_hbm, src_hbm, dst_hbm, *rest):
    no = len(out_type)
    outr_hbm = rest[0]
    outs_hbm = rest[1] if do_scal else None
    sc = list(rest[no:])
    sidx = sc[:NI]
    didx = sc[NI:2 * NI]
    rows = sc[2 * NI:2 * NI + NB]
    svals = sc[2 * NI + NB:2 * NI + 2 * NB]
    zscal = sc[2 * NI + 2 * NB]
    acc = sc[2 * NI + 2 * NB + 1]
    sacc = sc[2 * NI + 2 * NB + 2]
    sems_all = sc[2 * NI + 2 * NB + 3:]
    semi = sems_all[:NI]
    semg = sems_all[NI:NI + NB]
    sems = sems_all[NI + NB:]

    c = lax.axis_index("c")
    s = lax.axis_index("s")
    z16 = jnp.zeros((16,), jnp.float32)

    def zrow(r, _):
      for j in range(D // 16):
        rows[0][r, pl.ds(j * 16, 16)] = z16
      return 0

    lax.fori_loop(0, CHUNK, zrow, 0)

    def zsc(r, _):
      zscal[pl.ds(r * 16, 16)] = z16
      return 0

    lax.fori_loop(0, RPT // 16, zsc, 0)
    zscal[pl.ds(RPT - 16, 16)] = z16

    row0 = s * RPT
    rem = RPT % CHUNK
    nzf = RPT // CHUNK

    def zcpy(k):
      sz = CHUNK if k < nzf else rem
      zsrc = rows[0] if sz == CHUNK else rows[0].at[pl.ds(0, sz)]
      return pltpu.make_async_copy(zsrc, acc.at[pl.ds(row0 + k * CHUNK, sz)],
                                   sems[0])

    npieces = nzf + (1 if rem else 0)
    for k in range(npieces):
      zcpy(k).start()
    if do_scal:
      pltpu.make_async_copy(zscal, sacc.at[pl.ds(row0, RPT)], sems[1]).start()
    for k in range(npieces):
      zcpy(k).wait()
    if do_scal:
      pltpu.make_async_copy(zscal, sacc.at[pl.ds(row0, RPT)], sems[1]).wait()
    plsc.subcore_barrier()

    base = (s * NC + c) * EPT

    def idx_load(j, islot):
      off = base + j * CHUNK
      pltpu.make_async_copy(src_hbm.at[pl.ds(off, CHUNK)], sidx[islot],
                            semi[islot]).start()
      pltpu.make_async_copy(dst_hbm.at[pl.ds(off, CHUNK)], didx[islot],
                            semi[islot]).start()

    def idx_wait(islot):
      pltpu.make_async_copy(src_hbm.at[pl.ds(0, CHUNK)], sidx[islot],
                            semi[islot]).wait()
      pltpu.make_async_copy(dst_hbm.at[pl.ds(0, CHUNK)], didx[islot],
                            semi[islot]).wait()

    def gath(b, islot):
      return (pltpu.make_async_copy(table_hbm.at[sidx[islot]], rows[b], semg[b]),
              pltpu.make_async_copy(stab_hbm.at[sidx[islot]], svals[b], semg[b])
              if do_scal else None)

    def scat(b, islot):
      return (pltpu.make_async_copy(rows[b], acc.at[didx[islot]], sems[b]),
              pltpu.make_async_copy(svals[b], sacc.at[didx[islot]], sems[b])
              if do_scal else None)


    for b in range(NB):
      idx_load(b, b)

    def two_groups(gg, _):
      for half in range(2):
        g = gg * 2 + half
        lo = half * NB
        hi = (1 - half) * NB

        for b in range(NB):
          jb = g * NB + b

          @pl.when(g > 0)
          def _():
            ra, sa = scat(b, hi + b)
            ra.wait()
            if do_scal:
              sa.wait()

          idx_load(jb + NB, hi + b)

        for b in range(NB):
          idx_wait(lo + b)
          ra, sa = gath(b, lo + b)
          ra.start()
          if do_scal:
            sa.start()

        for b in range(NB):
          ra, sa = gath(b, lo + b)
          ra.wait()
          if do_scal:
            sa.wait()
          rb, sb = scat(b, lo + b)
          rb.start(add=True)
          if do_scal:
            sb.start(add=True)
      return 0

    lax.fori_loop(0, NGF // 2, two_groups, 0)

    for b in range(NB):
      ra, sa = scat(b, NB + b)
      ra.wait()
      if do_scal:
        sa.wait()
    for b in range(NB):
      idx_wait(b)
      ra, sa = gath(b, b)
      ra.start()
      if do_scal:
        sa.start()
    for b in range(NB):
      ra, sa = gath(b, b)
      ra.wait()
      if do_scal:
        sa.wait()
      rb, sb = scat(b, b)
      rb.start(add=True)
      if do_scal:
        sb.start(add=True)
    for b in range(NB):
      ra, sa = scat(b, b)
      ra.wait()
      if do_scal:
        sa.wait()

    for j in range((NGF + 1) * NB, NCHUNK):
      off = base + j * CHUNK
      pltpu.sync_copy(src_hbm.at[pl.ds(off, CHUNK)], sidx[NB])
      pltpu.sync_copy(dst_hbm.at[pl.ds(off, CHUNK)], didx[NB])
      ra, sa = gath(0, NB)
      ra.start()
      if do_scal:
        sa.start()
      ra.wait()
      if do_scal:
        sa.wait()
      rb, sb = scat(0, NB)
      rb.start(add=True)
      if do_scal:
        sb.start(add=True)
      rb.wait()
      if do_scal:
        sb.wait()

    plsc.subcore_barrier()
    def po_read(k):
      sz = CHUNK if k < nzf else rem
      b = k % NB
      dstb = rows[b] if sz == CHUNK else rows[b].at[pl.ds(0, sz)]
      return pltpu.make_async_copy(acc.at[pl.ds(row0 + k * CHUNK, sz)],
                                   dstb, semg[b])

    def po_write(k):
      sz = CHUNK if k < nzf else rem
      b = k % NB
      srcb = rows[b] if sz == CHUNK else rows[b].at[pl.ds(0, sz)]
      return pltpu.make_async_copy(srcb,
                                   outr_hbm.at[c, pl.ds(row0 + k * CHUNK, sz)],
                                   sems[b])

    if do_scal:
      pltpu.make_async_copy(sacc.at[pl.ds(row0, RPT)], zscal, semi[0]).start()
    for k in range(npieces):
      if k - NB >= 0:
        po_write(k - NB).wait()
      po_read(k).start()
      if k - 1 >= 0:
        po_read(k - 1).wait()
        po_write(k - 1).start()
    po_read(npieces - 1).wait()
    po_write(npieces - 1).start()
    for p in range(max(0, npieces - NB), npieces):
      po_write(p).wait()
    if do_scal:
      pltpu.make_async_copy(sacc.at[pl.ds(row0, RPT)], zscal, semi[0]).wait()
      pltpu.sync_copy(zscal, outs_hbm.at[pl.ds(c * NPAD + row0, RPT)])

  if do_scal:
    outr, outs = seg_kernel(table, stab, src, dst)
    return outr, outs.reshape(NC, NPAD)[:, :N]
  (outr,) = seg_kernel(table, stab, src, dst)
  return outr, None


def _embed_body(x_ref, pe_ref, wpe_ref, h_ref):
  h_ref[...] = x_ref[...] + jnp.dot(pe_ref[...], wpe_ref[...],
                                    preferred_element_type=jnp.float32)


def _gates_body(h_ref, sr_ref, ss_ref, ains_ref, ainn_ref, aino_ref, ainb_ref,
                aouts_ref, aoutn_ref, aouto_ref, aoutb_ref,
                ik_ref, ok_ref, gp_ref):
  h = h_ref[...]
  S = sr_ref[0, :N] + sr_ref[1, :N]
  deg = jnp.maximum(ss_ref[0] + ss_ref[1], 1.0)[:, None]
  m = S / deg

  def keep(a_s, a_n, a_o, a_b):
    z = jnp.maximum(jnp.dot(h, a_s, preferred_element_type=jnp.float32)
                    + jnp.dot(m, a_n, preferred_element_type=jnp.float32), 0.0)
    lo = jnp.dot(z, a_o, preferred_element_type=jnp.float32) + a_b
    return jax.nn.sigmoid((lo[:, 0:1] - lo[:, 1:2]) / TEMP)

  in_keep = keep(ains_ref[...], ainn_ref[...], aino_ref[...], ainb_ref[...])
  out_keep = keep(aouts_ref[...], aoutn_ref[...], aouto_ref[...], aoutb_ref[...])
  ik_ref[...] = in_keep
  ok_ref[...] = out_keep[:, 0]
  gp_ref[...] = out_keep * h


def _update_body(h_ref, sr_ref, ss_ref, ik_ref, ws_ref, wn_ref, b_ref, ho_ref):
  h = h_ref[...]
  S = sr_ref[0, :N] + sr_ref[1, :N]
  sw = (ss_ref[0] + ss_ref[1])[:, None]
  ik = ik_ref[...]
  m = (ik * S) / (ik * sw + 1e-8)
  ho_ref[...] = jnp.maximum(jnp.dot(h, ws_ref[...], preferred_element_type=jnp.float32)
                            + jnp.dot(m, wn_ref[...], preferred_element_type=jnp.float32)
                            + b_ref[...], 0.0) + h


def _tc(body, out_shape):
  return pl.pallas_call(body, out_shape=out_shape)


def kernel(x, edge_index, pestat, W_pe, W_self, W_nbr, b_env,
           Ain_self, Ain_nbr, Ain_out, Ain_bias,
           Aout_self, Aout_nbr, Aout_out, Aout_bias):
  src = edge_index[0].astype(jnp.int32)
  dst = edge_index[1].astype(jnp.int32)
  L = W_self.shape[0]

  h = _tc(_embed_body, jax.ShapeDtypeStruct((N, D), jnp.float32))(
      x, pestat, W_pe)

  ones_tab = jnp.ones((N,), jnp.float32)
  ainb = Ain_bias.reshape(1, 2)
  aoutb = Aout_bias.reshape(1, 2)

  deg2 = None
  for l in range(L):
    s0r, s0s = _seg_sum_partials(h, ones_tab, src, dst, do_scal=(l == 0))
    if l == 0:
      deg2 = s0s
    in_keep, ok_tab, gp = _tc(
        _gates_body,
        [jax.ShapeDtypeStruct((N, 1), jnp.float32),
         jax.ShapeDtypeStruct((N,), jnp.float32),
         jax.ShapeDtypeStruct((N, D), jnp.float32)],
    )(h, s0r, deg2, Ain_self, Ain_nbr, Ain_out, ainb,
      Aout_self, Aout_nbr, Aout_out, aoutb)
    s1r, s1s = _seg_sum_partials(gp, ok_tab, src, dst)
    h = _tc(_update_body, jax.ShapeDtypeStruct((N, D), jnp.float32))(
        h, s1r, s1s, in_keep, W_self[l], W_nbr[l], b_env[l].reshape(1, D))

  return h


# speedup vs baseline: 19.6603x; 1.0145x over previous
import functools

import jax
import jax.numpy as jnp
from jax import lax
from jax.experimental import pallas as pl
from jax.experimental.pallas import tpu as pltpu
from jax.experimental.pallas import tpu_sc as plsc

N = 10000
E = 320000
D = 128
H = 16
TEMP = 0.5

NC = 2
NS = 16
NW = NC * NS
EPT = E // NW
CHUNK = 80
NCHUNK = EPT // CHUNK
NPAD = 10112
RPT = NPAD // NS

NB = 4
NI = 2 * NB
NGF = 30


def _seg_sum_partials(table, stab, src, dst, do_scal=True, ones_scal=False):
  mesh = plsc.VectorSubcoreMesh(core_axis_name="c", subcore_axis_name="s")

  out_type = [jax.ShapeDtypeStruct((NC, NPAD, D), jnp.float32)]
  if do_scal:
    out_type.append(jax.ShapeDtypeStruct((NC * NPAD,), jnp.float32))

  scratch = (
      [pltpu.VMEM((CHUNK,), jnp.int32) for _ in range(NI)]
      + [pltpu.VMEM((CHUNK,), jnp.int32) for _ in range(NI)]
      + [pltpu.VMEM((CHUNK, D), jnp.float32) for _ in range(NB)]
      + [pltpu.VMEM((CHUNK,), jnp.float32) for _ in range(NB)]
      + [pltpu.VMEM((RPT,), jnp.float32)]
      + [pltpu.VMEM_SHARED((NPAD, D), jnp.float32)]
      + [pltpu.VMEM_SHARED((NPAD,), jnp.float32)]
      + [pltpu.SemaphoreType.DMA for _ in range(NI + 2 * NB)]
  )

  @functools.partial(pl.kernel, out_type=tuple(out_type), mesh=mesh,
                     scratch_types=tuple(scratch))
  def seg_kernel(table_hbm, stab_hbm, src_hbm, dst_hbm, *rest):
    no = len(out_type)
    outr_hbm = rest[0]
    outs_hbm = rest[1] if do_scal else None
    sc = list(rest[no:])
    sidx = sc[:NI]
    didx = sc[NI:2 * NI]
    rows = sc[2 * NI:2 * NI + NB]
    svals = sc[2 * NI + NB:2 * NI + 2 * NB]
    zscal = sc[2 * NI + 2 * NB]
    acc = sc[2 * NI + 2 * NB + 1]
    sacc = sc[2 * NI + 2 * NB + 2]
    sems_all = sc[2 * NI + 2 * NB + 3:]
    semi = sems_all[:NI]
    semg = sems_all[NI:NI + NB]
    sems = sems_all[NI + NB:]

    c = lax.axis_index("c")
    s = lax.axis_index("s")
    z16 = jnp.zeros((16,), jnp.float32)

    def zrow(r, _):
      for j in range(D // 16):
        rows[0][r, pl.ds(j * 16, 16)] = z16
      return 0

    lax.fori_loop(0, CHUNK, zrow, 0)

    def zsc(r, _):
      zscal[pl.ds(r * 16, 16)] = z16
      return 0

    lax.fori_loop(0, RPT // 16, zsc, 0)
    zscal[pl.ds(RPT - 16, 16)] = z16

    if do_scal and ones_scal:
      one16 = jnp.ones((16,), jnp.float32)
      for b in range(NB):
        for j in range(CHUNK // 16):
          svals[b][pl.ds(j * 16, 16)] = one16

    row0 = s * RPT
    rem = RPT % CHUNK
    nzf = RPT // CHUNK

    def zcpy(k):
      sz = CHUNK if k < nzf else rem
      zsrc = rows[0] if sz == CHUNK else rows[0].at[pl.ds(0, sz)]
      return pltpu.make_async_copy(zsrc, acc.at[pl.ds(row0 + k * CHUNK, sz)],
                                   sems[0])

    npieces = nzf + (1 if rem else 0)
    for k in range(npieces):
      zcpy(k).start()
    if do_scal:
      pltpu.make_async_copy(zscal, sacc.at[pl.ds(row0, RPT)], sems[1]).start()
    for k in range(npieces):
      zcpy(k).wait()
    if do_scal:
      pltpu.make_async_copy(zscal, sacc.at[pl.ds(row0, RPT)], sems[1]).wait()
    plsc.subcore_barrier()

    base = (s * NC + c) * EPT

    def idx_load(j, islot):
      off = base + j * CHUNK
      pltpu.make_async_copy(src_hbm.at[pl.ds(off, CHUNK)], sidx[islot],
                            semi[islot]).start()
      pltpu.make_async_copy(dst_hbm.at[pl.ds(off, CHUNK)], didx[islot],
                            semi[islot]).start()

    def idx_wait(islot):
      pltpu.make_async_copy(src_hbm.at[pl.ds(0, CHUNK)], sidx[islot],
                            semi[islot]).wait()
      pltpu.make_async_copy(dst_hbm.at[pl.ds(0, CHUNK)], didx[islot],
                            semi[islot]).wait()

    gath_scal = do_scal and not ones_scal

    def gath(b, islot):
      return (pltpu.make_async_copy(table_hbm.at[sidx[islot]], rows[b], semg[b]),
              pltpu.make_async_copy(stab_hbm.at[sidx[islot]], svals[b], semg[b])
              if gath_scal else None)

    def scat(b, islot):
      return (pltpu.make_async_copy(rows[b], acc.at[didx[islot]], sems[b]),
              pltpu.make_async_copy(svals[b], sacc.at[didx[islot]], sems[b])
              if do_scal else None)


    for b in range(NB):
      idx_load(b, b)

    def two_groups(gg, _):
      for half in range(2):
        g = gg * 2 + half
        lo = half * NB
        hi = (1 - half) * NB

        for b in range(NB):
          jb = g * NB + b

          @pl.when(g > 0)
          def _():
            ra, sa = scat(b, hi + b)
            ra.wait()
            if do_scal:
              sa.wait()

          idx_load(jb + NB, hi + b)

        for b in range(NB):
          idx_wait(lo + b)
          ra, sa = gath(b, lo + b)
          ra.start()
          if gath_scal:
            sa.start()

        for b in range(NB):
          ra, sa = gath(b, lo + b)
          ra.wait()
          if gath_scal:
            sa.wait()
          rb, sb = scat(b, lo + b)
          rb.start(add=True)
          if do_scal:
            sb.start(add=True)
      return 0

    lax.fori_loop(0, NGF // 2, two_groups, 0)

    for b in range(NB):
      ra, sa = scat(b, NB + b)
      ra.wait()
      if do_scal:
        sa.wait()
    for b in range(NB):
      idx_wait(b)
      ra, sa = gath(b, b)
      ra.start()
      if gath_scal:
        sa.start()
    for b in range(NB):
      ra, sa = gath(b, b)
      ra.wait()
      if gath_scal:
        sa.wait()
      rb, sb = scat(b, b)
      rb.start(add=True)
      if do_scal:
        sb.start(add=True)
    for b in range(NB):
      ra, sa = scat(b, b)
      ra.wait()
      if do_scal:
        sa.wait()

    for j in range((NGF + 1) * NB, NCHUNK):
      off = base + j * CHUNK
      pltpu.sync_copy(src_hbm.at[pl.ds(off, CHUNK)], sidx[NB])
      pltpu.sync_copy(dst_hbm.at[pl.ds(off, CHUNK)], didx[NB])
      ra, sa = gath(0, NB)
      ra.start()
      if gath_scal:
        sa.start()
      ra.wait()
      if gath_scal:
        sa.wait()
      rb, sb = scat(0, NB)
      rb.start(add=True)
      if do_scal:
        sb.start(add=True)
      rb.wait()
      if do_scal:
        sb.wait()

    plsc.subcore_barrier()
    def po_read(k):
      sz = CHUNK if k < nzf else rem
      b = k % NB
      dstb = rows[b] if sz == CHUNK else rows[b].at[pl.ds(0, sz)]
      return pltpu.make_async_copy(acc.at[pl.ds(row0 + k * CHUNK, sz)],
                                   dstb, semg[b])

    def po_write(k):
      sz = CHUNK if k < nzf else rem
      b = k % NB
      srcb = rows[b] if sz == CHUNK else rows[b].at[pl.ds(0, sz)]
      return pltpu.make_async_copy(srcb,
                                   outr_hbm.at[c, pl.ds(row0 + k * CHUNK, sz)],
                                   sems[b])

    if do_scal:
      pltpu.make_async_copy(sacc.at[pl.ds(row0, RPT)], zscal, semi[0]).start()
    for k in range(npieces):
      if k - NB >= 0:
        po_write(k - NB).wait()
      po_read(k).start()
      if k - 1 >= 0:
        po_read(k - 1).wait()
        po_write(k - 1).start()
    po_read(npieces - 1).wait()
    po_write(npieces - 1).start()
    for p in range(max(0, npieces - NB), npieces):
      po_write(p).wait()
    if do_scal:
      pltpu.make_async_copy(sacc.at[pl.ds(row0, RPT)], zscal, semi[0]).wait()
      pltpu.sync_copy(zscal, outs_hbm.at[pl.ds(c * NPAD + row0, RPT)])

  if do_scal:
    outr, outs = seg_kernel(table, stab, src, dst)
    return outr, outs.reshape(NC, NPAD)[:, :N]
  (outr,) = seg_kernel(table, stab, src, dst)
  return outr, None


def _embed_body(x_ref, pe_ref, wpe_ref, h_ref):
  h_ref[...] = x_ref[...] + jnp.dot(pe_ref[...], wpe_ref[...],
                                    preferred_element_type=jnp.float32)


def _gates_body(h_ref, sr_ref, ss_ref, ains_ref, ainn_ref, aino_ref, ainb_ref,
                aouts_ref, aoutn_ref, aouto_ref, aoutb_ref,
                ik_ref, ok_ref, gp_ref):
  h = h_ref[...]
  S = sr_ref[0, :N] + sr_ref[1, :N]
  deg = jnp.maximum(ss_ref[0] + ss_ref[1], 1.0)[:, None]
  m = S / deg

  def keep(a_s, a_n, a_o, a_b):
    z = jnp.maximum(jnp.dot(h, a_s, preferred_element_type=jnp.float32)
                    + jnp.dot(m, a_n, preferred_element_type=jnp.float32), 0.0)
    lo = jnp.dot(z, a_o, preferred_element_type=jnp.float32) + a_b
    return jax.nn.sigmoid((lo[:, 0:1] - lo[:, 1:2]) / TEMP)

  in_keep = keep(ains_ref[...], ainn_ref[...], aino_ref[...], ainb_ref[...])
  out_keep = keep(aouts_ref[...], aoutn_ref[...], aouto_ref[...], aoutb_ref[...])
  ik_ref[...] = in_keep
  ok_ref[...] = out_keep[:, 0]
  gp_ref[...] = out_keep * h


def _update_body(h_ref, sr_ref, ss_ref, ik_ref, ws_ref, wn_ref, b_ref, ho_ref):
  h = h_ref[...]
  S = sr_ref[0, :N] + sr_ref[1, :N]
  sw = (ss_ref[0] + ss_ref[1])[:, None]
  ik = ik_ref[...]
  m = (ik * S) / (ik * sw + 1e-8)
  ho_ref[...] = jnp.maximum(jnp.dot(h, ws_ref[...], preferred_element_type=jnp.float32)
                            + jnp.dot(m, wn_ref[...], preferred_element_type=jnp.float32)
                            + b_ref[...], 0.0) + h


def _tc(body, out_shape):
  return pl.pallas_call(body, out_shape=out_shape)


def kernel(x, edge_index, pestat, W_pe, W_self, W_nbr, b_env,
           Ain_self, Ain_nbr, Ain_out, Ain_bias,
           Aout_self, Aout_nbr, Aout_out, Aout_bias):
  src = edge_index[0].astype(jnp.int32)
  dst = edge_index[1].astype(jnp.int32)
  L = W_self.shape[0]

  h = _tc(_embed_body, jax.ShapeDtypeStruct((N, D), jnp.float32))(
      x, pestat, W_pe)

  ones_tab = jnp.ones((N,), jnp.float32)
  ainb = Ain_bias.reshape(1, 2)
  aoutb = Aout_bias.reshape(1, 2)

  deg2 = None
  for l in range(L):
    s0r, s0s = _seg_sum_partials(h, ones_tab, src, dst, do_scal=(l == 0),
                                 ones_scal=True)
    if l == 0:
      deg2 = s0s
    in_keep, ok_tab, gp = _tc(
        _gates_body,
        [jax.ShapeDtypeStruct((N, 1), jnp.float32),
         jax.ShapeDtypeStruct((N,), jnp.float32),
         jax.ShapeDtypeStruct((N, D), jnp.float32)],
    )(h, s0r, deg2, Ain_self, Ain_nbr, Ain_out, ainb,
      Aout_self, Aout_nbr, Aout_out, aoutb)
    s1r, s1s = _seg_sum_partials(gp, ok_tab, src, dst)
    h = _tc(_update_body, jax.ShapeDtypeStruct((N, D), jnp.float32))(
        h, s1r, s1s, in_keep, W_self[l], W_nbr[l], b_env[l].reshape(1, D))

  return h
